# Initial kernel scaffold; baseline (speedup 1.0000x reference)
import jax
import jax.numpy as jnp
from jax.experimental import pallas as pl


def kernel(var_feats, soc_feats, con_feats, src_v2c, dst_v2c, src_s2c, dst_s2c, src_c2v, dst_c2v, src_c2s, dst_c2s, ew_v2c, ew_s2c, ew_c2v, ew_c2s, W_var, b_var, W_soc, b_soc, W_con, b_con, W1, b1, W2, b2, Wo1, bo1, Wo2, bo2, Wo3, bo3):
    raise NotImplementedError("write your pallas kernel here")


# speedup vs baseline: 9.9634x; 9.9634x over previous
import functools

import jax
import jax.numpy as jnp
from jax import lax
from jax.experimental import pallas as pl
from jax.experimental.pallas import tpu as pltpu
from jax.experimental.pallas import tpu_sc as plsc

NV = 50000
NS_ = 10000
NCN = 40000
NV_P = 51200
NCN_P = 40960
L = 16
NCORE = 2
NSUB = 16
NW = NCORE * NSUB

_HOFF = (0, NV, NV + NCN, NV + NCN + NS_, NV + 2 * NCN + NS_, NV + 3 * NCN + NS_)
HIST_VALID = 2 * NV + 3 * NCN + NS_
HIST_N = 230400


def _sc_histogram(di2, ones_row, zhist):
    rows = di2.shape[0]
    rpt = rows // NW
    KH = 8
    nblk = rpt // KH
    slc = HIST_N // NSUB
    mesh = plsc.VectorSubcoreMesh(core_axis_name="c", subcore_axis_name="s")

    zc = slc // 8

    @functools.partial(
        pl.kernel, mesh=mesh,
        out_type=jax.ShapeDtypeStruct((NCORE * HIST_N,), jnp.float32),
        compiler_params=pltpu.CompilerParams(
            needs_layout_passes=False, use_tc_tiling_on_sc=False),
        scratch_types=[
            pltpu.VMEM((KH, 128), jnp.int32),
            pltpu.VMEM((128,), jnp.float32),
            pltpu.VMEM((slc // 8,), jnp.float32),
            pltpu.VMEM_SHARED((HIST_N,), jnp.float32),
            pltpu.SemaphoreType.DMA,
        ],
    )
    def k(di_hbm, ones_hbm, z_hbm, out_hbm, idx_v, ones_v, z_v, hist_s, sem):
        cid = lax.axis_index("c")
        sid = lax.axis_index("s")
        wid = sid * NCORE + cid
        pltpu.sync_copy(ones_hbm, ones_v)
        pltpu.sync_copy(z_hbm, z_v)
        for q in range(8):
            pltpu.sync_copy(z_v, hist_s.at[pl.ds(sid * slc + q * zc, zc)])
        plsc.subcore_barrier()

        def blk(b, carry):
            row0 = wid * rpt + b * KH
            pltpu.sync_copy(di_hbm.at[pl.ds(row0, KH)], idx_v)
            cps = [pltpu.async_copy(ones_v, hist_s.at[idx_v.at[j]], sem, add=True)
                   for j in range(KH)]
            for c in cps:
                c.wait()
            return carry

        lax.fori_loop(0, nblk, blk, 0)
        plsc.subcore_barrier()
        for q in range(8):
            pltpu.sync_copy(hist_s.at[pl.ds(sid * slc + q * zc, zc)], z_v)
            pltpu.sync_copy(
                z_v, out_hbm.at[pl.ds(cid * HIST_N + sid * slc + q * zc, zc)])

    return k(di2, ones_row, zhist).reshape(NCORE, HIST_N)


def _sc_edge_pass(table, src2, dst2, ew2, zrows, n_dst_p, kb):
    rows = src2.shape[0]
    rpt = rows // NW
    nblk = rpt // kb
    zr = n_dst_p // NSUB
    mesh = plsc.VectorSubcoreMesh(core_axis_name="c", subcore_axis_name="s")

    @functools.partial(
        pl.kernel, mesh=mesh,
        out_type=jax.ShapeDtypeStruct((NCORE, n_dst_p, L), jnp.float32),
        compiler_params=pltpu.CompilerParams(
            needs_layout_passes=False, use_tc_tiling_on_sc=False),
        scratch_types=[
            pltpu.VMEM((kb, 128), jnp.int32),
            pltpu.VMEM((kb, 128), jnp.int32),
            pltpu.VMEM((kb, 128), jnp.float32),
            pltpu.VMEM((kb, 128, L), jnp.float32),
            pltpu.VMEM((zr // 8, L), jnp.float32),
            pltpu.VMEM_SHARED((n_dst_p, L), jnp.float32),
            pltpu.SemaphoreType.DMA,
        ],
    )
    def k(tab_hbm, s_hbm, d_hbm, e_hbm, z_hbm, out_hbm,
          s_v, d_v, e_v, r_v, z_v, acc, sem):
        cid = lax.axis_index("c")
        sid = lax.axis_index("s")
        wid = sid * NCORE + cid
        pltpu.sync_copy(z_hbm, z_v)
        zc = zr // 8
        for q in range(8):
            pltpu.sync_copy(z_v, acc.at[pl.ds(sid * zr + q * zc, zc)])
        plsc.subcore_barrier()

        lanes = lax.iota(jnp.int32, L)

        def blk(b, carry):
            row0 = wid * rpt + b * kb
            pltpu.sync_copy(s_hbm.at[pl.ds(row0, kb)], s_v)
            pltpu.sync_copy(d_hbm.at[pl.ds(row0, kb)], d_v)
            pltpu.sync_copy(e_hbm.at[pl.ds(row0, kb)], e_v)
            gs = [pltpu.async_copy(tab_hbm.at[s_v.at[j]], r_v.at[j], sem)
                  for j in range(kb)]
            for g in gs:
                g.wait()

            def scale(e, carry2):
                fe = jnp.full((L,), e, jnp.int32)
                for j in range(kb):
                    fj = jnp.full((L,), j, jnp.int32)
                    row = plsc.load_gather(r_v, [fj, fe, lanes])
                    wb = plsc.load_gather(e_v, [fj, fe])
                    plsc.store_scatter(r_v, [fj, fe, lanes], row * wb)
                return carry2

            lax.fori_loop(0, 128, scale, 0)
            for j in range(kb):
                pltpu.sync_copy(r_v.at[j], acc.at[d_v.at[j]], add=True)
            return carry

        lax.fori_loop(0, nblk, blk, 0)
        plsc.subcore_barrier()
        for q in range(8):
            pltpu.sync_copy(acc.at[pl.ds(sid * zr + q * zc, zc)], z_v)
            pltpu.sync_copy(z_v, out_hbm.at[cid, pl.ds(sid * zr + q * zc, zc)])

    return k(table, src2, dst2, ew2, zrows)


def _tc_degree_scales(hp):
    r = HIST_N // 128
    h3 = hp.reshape(NCORE, r, 128)

    def body(h_ref, o_ref):
        o_ref[...] = lax.rsqrt(jnp.maximum(h_ref[0] + h_ref[1], 1.0))

    out = pl.pallas_call(
        body,
        out_shape=jax.ShapeDtypeStruct((r, 128), jnp.float32),
    )(h3)
    return out.reshape(HIST_N)


def _tc_table(x, w, b, so_col, bm):
    n = x.shape[0]
    nb = n // bm

    def body(x_ref, w_ref, b_ref, s_ref, o_ref):
        o_ref[...] = jnp.maximum(
            jnp.dot(x_ref[...], w_ref[...],
                    preferred_element_type=jnp.float32) + b_ref[...],
            0.0) * s_ref[...]

    return pl.pallas_call(
        body,
        grid=(nb,),
        in_specs=[pl.BlockSpec((bm, x.shape[1]), lambda i: (i, 0)),
                  pl.BlockSpec((x.shape[1], L), lambda i: (0, 0)),
                  pl.BlockSpec((1, L), lambda i: (0, 0)),
                  pl.BlockSpec((bm, 1), lambda i: (i, 0))],
        out_specs=pl.BlockSpec((bm, L), lambda i: (i, 0)),
        out_shape=jax.ShapeDtypeStruct((n, L), jnp.float32),
    )(x, w, b, so_col)


def _tc_mid(pv, ps, siv, sis, soc, w2, bias, bm):
    nb = NCN_P // bm

    def body(a_ref, s_ref, siv_ref, sis_ref, soc_ref, w_ref, b_ref, o_ref):
        s = ((a_ref[0] + a_ref[1]) * siv_ref[...]
             + (s_ref[0] + s_ref[1]) * sis_ref[...])
        o_ref[...] = jnp.maximum(
            jnp.dot(s, w_ref[...], preferred_element_type=jnp.float32)
            + b_ref[...], 0.0) * soc_ref[...]

    return pl.pallas_call(
        body,
        grid=(nb,),
        in_specs=[pl.BlockSpec((NCORE, bm, L), lambda i: (0, i, 0)),
                  pl.BlockSpec((NCORE, bm, L), lambda i: (0, i, 0)),
                  pl.BlockSpec((bm, 1), lambda i: (i, 0)),
                  pl.BlockSpec((bm, 1), lambda i: (i, 0)),
                  pl.BlockSpec((bm, 1), lambda i: (i, 0)),
                  pl.BlockSpec((L, L), lambda i: (0, 0)),
                  pl.BlockSpec((1, L), lambda i: (0, 0))],
        out_specs=pl.BlockSpec((bm, L), lambda i: (i, 0)),
        out_shape=jax.ShapeDtypeStruct((NCN_P, L), jnp.float32),
    )(pv, ps, siv, sis, soc, w2, bias)


def _tc_final(q, sic, w2, b2, wo1, bo1, wo2, bo2, w3row, b3, bm):
    nb = NV_P // bm

    def body(q_ref, sic_ref, w2_ref, b2_ref, wo1_ref, bo1_ref, wo2_ref,
             bo2_ref, w3_ref, b3_ref, o_ref, acc_ref):
        i = pl.program_id(0)

        @pl.when(i == 0)
        def _():
            acc_ref[...] = jnp.zeros_like(acc_ref)

        s = (q_ref[0] + q_ref[1]) * sic_ref[...]
        hv = jnp.maximum(
            jnp.dot(s, w2_ref[...], preferred_element_type=jnp.float32)
            + b2_ref[...], 0.0)
        h1 = jnp.maximum(
            jnp.dot(hv, wo1_ref[...], preferred_element_type=jnp.float32)
            + bo1_ref[...], 0.0)
        h2 = jnp.maximum(
            jnp.dot(h1, wo2_ref[...], preferred_element_type=jnp.float32)
            + bo2_ref[...], 0.0)
        rowid = i * bm + lax.broadcasted_iota(jnp.int32, (bm, 1), 0)
        h2 = jnp.where(rowid < NV, h2, 0.0)
        acc_ref[0:1, 0:L] += jnp.sum(h2, axis=0, keepdims=True)

        @pl.when(i == nb - 1)
        def _():
            t = jnp.sum(acc_ref[0:1, 0:L] * w3_ref[...], axis=1, keepdims=True)
            o_ref[...] = t / float(NV) + b3_ref[...]

    return pl.pallas_call(
        body,
        grid=(nb,),
        in_specs=[pl.BlockSpec((NCORE, bm, L), lambda i: (0, i, 0)),
                  pl.BlockSpec((bm, 1), lambda i: (i, 0)),
                  pl.BlockSpec((L, L), lambda i: (0, 0)),
                  pl.BlockSpec((1, L), lambda i: (0, 0)),
                  pl.BlockSpec((L, L), lambda i: (0, 0)),
                  pl.BlockSpec((1, L), lambda i: (0, 0)),
                  pl.BlockSpec((L, L), lambda i: (0, 0)),
                  pl.BlockSpec((1, L), lambda i: (0, 0)),
                  pl.BlockSpec((1, L), lambda i: (0, 0)),
                  pl.BlockSpec((1, 1), lambda i: (0, 0))],
        out_specs=pl.BlockSpec((1, 1), lambda i: (0, 0)),
        out_shape=jax.ShapeDtypeStruct((1, 1), jnp.float32),
        scratch_shapes=[pltpu.VMEM((8, 128), jnp.float32)],
    )(q, sic, w2, b2, wo1, bo1, wo2, bo2, w3row, b3)


def _pad_edges(src, dst, ew, rows, n_src, n_dst):
    e = src.shape[0]
    p = rows * 128 - e
    ar = jnp.arange(p, dtype=jnp.int32)
    src_p = jnp.concatenate([src.astype(jnp.int32), ar % n_src]).reshape(rows, 128)
    dst_p = jnp.concatenate([dst.astype(jnp.int32), ar % n_dst]).reshape(rows, 128)
    ew_p = jnp.concatenate([ew, jnp.zeros((p,), jnp.float32)]).reshape(rows, 128)
    return src_p, dst_p, ew_p


def _padw(w, r, c):
    return jnp.pad(w, ((0, r - w.shape[0]), (0, c - w.shape[1])))


def kernel(var_feats, soc_feats, con_feats, src_v2c, dst_v2c, src_s2c, dst_s2c,
           src_c2v, dst_c2v, src_c2s, dst_c2s, ew_v2c, ew_s2c, ew_c2v, ew_c2s,
           W_var, b_var, W_soc, b_soc, W_con, b_con, W1, b1, W2, b2,
           Wo1, bo1, Wo2, bo2, Wo3, bo3):
    f32 = jnp.float32
    i32 = jnp.int32

    n_deg = 2 * (2 * src_v2c.shape[0] + src_s2c.shape[0])
    drows = 27648
    pad = drows * 128 - n_deg
    di = jnp.concatenate([
        src_v2c.astype(i32) + _HOFF[0],
        dst_v2c.astype(i32) + _HOFF[1],
        src_s2c.astype(i32) + _HOFF[2],
        dst_s2c.astype(i32) + _HOFF[3],
        src_c2v.astype(i32) + _HOFF[4],
        dst_c2v.astype(i32) + _HOFF[5],
        HIST_VALID + (jnp.arange(pad, dtype=i32) % (HIST_N - HIST_VALID)),
    ]).reshape(drows, 128)

    ones_row = jnp.ones((128,), f32)
    zhist = jnp.zeros((HIST_N // NSUB // 8,), f32)
    hp = _sc_histogram(di, ones_row, zhist)
    dsc = _tc_degree_scales(hp)

    def col(lo, hi, n_pad):
        c = dsc[lo:hi].reshape(-1, 1)
        return jnp.pad(c, ((0, n_pad - (hi - lo)), (0, 0)))

    so_v2c = col(_HOFF[0], _HOFF[1], NV)
    si_v2c = col(_HOFF[1], _HOFF[2], NCN_P)
    so_s2c = col(_HOFF[2], _HOFF[3], NS_)
    si_s2c = col(_HOFF[3], _HOFF[4], NCN_P)
    so_c2v = col(_HOFF[4], _HOFF[5], NCN_P)
    si_c2v = col(_HOFF[5], HIST_VALID, NV_P)

    Wv = _padw(W_var, 128, L)
    Ws = _padw(W_soc, 128, L)
    bv = jnp.pad(b_var, (0, L - b_var.shape[0])).reshape(1, L)
    bs = jnp.pad(b_soc, (0, L - b_soc.shape[0])).reshape(1, L)
    X_var = _tc_table(var_feats, Wv, bv, so_v2c, 2000)
    X_soc = _tc_table(soc_feats, Ws, bs, so_s2c, 2000)

    sv2, dv2, ev2 = _pad_edges(src_v2c, dst_v2c, ew_v2c, 6400, NV, NCN)
    ss2, ds2, es2 = _pad_edges(src_s2c, dst_s2c, ew_s2c, 1280, NS_, NCN)
    sc2, dc2, ec2 = _pad_edges(src_c2v, dst_c2v, ew_c2v, 6400, NCN, NV)
    z_con = jnp.zeros((NCN_P // NSUB // 8, L), f32)
    z_var = jnp.zeros((NV_P // NSUB // 8, L), f32)

    pv = _sc_edge_pass(X_var, sv2, dv2, ev2, z_con, NCN_P, 8)
    ps = _sc_edge_pass(X_soc, ss2, ds2, es2, z_con, NCN_P, 8)

    W2p = _padw(W2, L, L)
    b2p = jnp.pad(b2, (0, L - b2.shape[0])).reshape(1, L)
    h_con = _tc_mid(pv, ps, si_v2c, si_s2c, so_c2v, W2p, 2.0 * b2p, 2560)

    pc = _sc_edge_pass(h_con, sc2, dc2, ec2, z_var, NV_P, 8)

    Wo1p = _padw(Wo1, L, L)
    Wo2p = _padw(Wo2, L, L)
    bo1p = jnp.pad(bo1, (0, L - bo1.shape[0])).reshape(1, L)
    bo2p = jnp.pad(bo2, (0, L - bo2.shape[0])).reshape(1, L)
    w3row = jnp.pad(Wo3[:, 0], (0, L - Wo3.shape[0])).reshape(1, L)
    b3 = bo3.reshape(1, 1)
    return _tc_final(pc, si_c2v, W2p, b2p, Wo1p, bo1p, Wo2p, bo2p, w3row, b3,
                     3200)
